# baseline (speedup 1.0000x reference)
import functools
import math

import jax
import jax.numpy as jnp
from jax import lax
from jax.experimental import pallas as pl
from jax.experimental.pallas import tpu as pltpu
from jax.experimental.pallas import tpu_sc as plsc

_NBUF = 2


@functools.cache
def _build(B, S, D, V):
    info = plsc.get_sparse_core_info()
    NC, NS, L = info.num_cores, info.num_subcores, info.num_lanes
    NW = NC * NS
    NB = B // NW
    scale = math.sqrt(float(D))
    n_outer = S // _NBUF

    mesh = plsc.VectorSubcoreMesh(core_axis_name="c", subcore_axis_name="s")

    @functools.partial(
        pl.kernel,
        out_type=jax.ShapeDtypeStruct((B, S * D), jnp.float32),
        mesh=mesh,
        scratch_types=[
            pltpu.VMEM((S, NB), jnp.int32),
            pltpu.VMEM((S, D), jnp.float32),
            [pltpu.VMEM((NB, D), jnp.float32) for _ in range(_NBUF)],
            [pltpu.VMEM((NB, D), jnp.float32) for _ in range(_NBUF)],
            [pltpu.SemaphoreType.DMA for _ in range(_NBUF)],
            [pltpu.SemaphoreType.DMA for _ in range(_NBUF)],
        ],
    )
    def emb_kernel(table_hbm, xt_hbm, enc_hbm, out_hbm,
                   idx_v, enc_v, gbufs, wbufs, gsems, wsems):
        wid = lax.axis_index("s") * NC + lax.axis_index("c")
        b0 = wid * NB
        pltpu.sync_copy(xt_hbm.at[:, pl.ds(b0, NB)], idx_v)
        pltpu.sync_copy(enc_hbm, enc_v)

        def gather(p, b):
            return pltpu.async_copy(table_hbm.at[idx_v.at[p]], gbufs[b], gsems[b])

        def writeout(p, b):
            return pltpu.async_copy(
                wbufs[b], out_hbm.at[pl.ds(b0, NB), pl.ds(p * D, D)], wsems[b])

        for b in range(_NBUF):
            gather(b, b)

        def outer(i, carry):
            for b in range(_NBUF):
                p = i * _NBUF + b
                pltpu.make_async_copy(
                    table_hbm.at[idx_v.at[p]], gbufs[b], gsems[b]).wait()
                @pl.when(i > 0)
                def _():
                    pltpu.make_async_copy(
                        wbufs[b],
                        out_hbm.at[pl.ds(b0, NB), pl.ds((p - _NBUF) * D, D)],
                        wsems[b]).wait()

                e = [enc_v[p, pl.ds(c * L, L)] for c in range(D // L)]

                def row_body(r, c2):
                    for c in range(D // L):
                        sl = pl.ds(c * L, L)
                        wbufs[b][r, sl] = gbufs[b][r, sl] * scale + e[c]
                    return c2

                lax.fori_loop(0, NB, row_body, 0, unroll=2)
                writeout(p, b)

                @pl.when(p + _NBUF < S)
                def _():
                    gather(p + _NBUF, b)
            return carry

        lax.fori_loop(0, n_outer, outer, 0)
        for b in range(_NBUF):
            p = S - _NBUF + b
            pltpu.make_async_copy(
                wbufs[b], out_hbm.at[pl.ds(b0, NB), pl.ds(p * D, D)],
                wsems[b]).wait()

    return emb_kernel


def kernel(x, table, encoding):
    B, S = x.shape
    V, D = table.shape
    xt = x.T.astype(jnp.int32)
    enc = encoding[:S, :]
    out = _build(B, S, D, V)(table, xt, enc)
    return out.reshape(B, S, D)


# speedup vs baseline: 4.5265x; 4.5265x over previous
import functools
import math

import jax
import jax.numpy as jnp
from jax import lax
from jax.experimental import pallas as pl
from jax.experimental.pallas import tpu as pltpu
from jax.experimental.pallas import tpu_sc as plsc

_NBUF = 5
_C = 128


@functools.cache
def _build(B, S, D, V):
    info = plsc.get_sparse_core_info()
    NC, NS, L = info.num_cores, info.num_subcores, info.num_lanes
    NW = NC * NS
    R = B * S
    rows_w = R // NW
    n_chunks = rows_w // _C
    n_outer = n_chunks // _NBUF
    scale = math.sqrt(float(D))

    mesh = plsc.VectorSubcoreMesh(core_axis_name="c", subcore_axis_name="s")

    @functools.partial(
        pl.kernel,
        out_type=jax.ShapeDtypeStruct((R, D), jnp.float32),
        mesh=mesh,
        scratch_types=[
            pltpu.VMEM((n_chunks, _C), jnp.int32),
            pltpu.MemorySpace.VMEM_SHARED((2 * S, D), jnp.float32),
            [pltpu.VMEM((_C, D), jnp.float32) for _ in range(_NBUF)],
            [pltpu.SemaphoreType.DMA for _ in range(_NBUF)],
            [pltpu.SemaphoreType.DMA for _ in range(_NBUF)],
            [pltpu.SemaphoreType.DMA for _ in range(_NBUF)],
        ],
    )
    def emb_kernel(table_hbm, x3_hbm, enc2_hbm, out_hbm,
                   idx_v, enc2_sh, bufs, psems, gsems, wsems):
        cid = lax.axis_index("c")
        sid = lax.axis_index("s")
        wid = sid * NC + cid
        row0 = wid * rows_w

        @pl.when(sid == 0)
        def _():
            pltpu.sync_copy(enc2_hbm, enc2_sh)
        pltpu.sync_copy(x3_hbm.at[wid], idx_v)
        plsc.subcore_barrier()

        def enc_src(p):
            off = lax.rem(p * _C, S)
            return enc2_sh.at[pl.ds(off, _C)]

        def out_dst(p):
            return out_hbm.at[pl.ds(row0 + p * _C, _C)]

        def prefill(p, b):
            pltpu.async_copy(enc_src(p), bufs[b], psems[b])

        def gather_add(p, b):
            pltpu.async_copy(table_hbm.at[idx_v.at[p]], bufs[b], gsems[b],
                             add=True)

        def wait_prefill(p, b):
            pltpu.make_async_copy(enc_src(p), bufs[b], psems[b]).wait()

        def wait_gather(p, b):
            pltpu.make_async_copy(table_hbm.at[idx_v.at[p]], bufs[b],
                                  gsems[b]).wait()

        def wait_writeout(p, b):
            pltpu.make_async_copy(bufs[b], out_dst(p), wsems[b]).wait()

        for p in range(3):
            prefill(p, p)
        for p in range(2):
            wait_prefill(p, p)
            gather_add(p, p)

        def outer(i, carry):
            for b in range(_NBUF):
                q = i * _NBUF + b
                qa = q + 3
                ba = (b + 3) % _NBUF

                @pl.when(qa < n_chunks)
                def _():
                    @pl.when(q >= 2)
                    def _():
                        wait_writeout(qa - _NBUF, ba)
                    prefill(qa, ba)

                qb = q + 2
                bb = (b + 2) % _NBUF

                @pl.when(qb < n_chunks)
                def _():
                    wait_prefill(qb, bb)
                    gather_add(qb, bb)

                wait_gather(q, b)

                def row_body(r, c2):
                    for c in range(D // L):
                        sl = pl.ds(c * L, L)
                        bufs[b][r, sl] = bufs[b][r, sl] * scale
                    return c2

                lax.fori_loop(0, _C, row_body, 0, unroll=2)
                pltpu.async_copy(bufs[b], out_dst(q), wsems[b])
            return carry

        lax.fori_loop(0, n_outer, outer, 0)
        for p in range(n_chunks - _NBUF, n_chunks):
            wait_writeout(p, p % _NBUF)

    return emb_kernel


def kernel(x, table, encoding):
    B, S = x.shape
    V, D = table.shape
    NW = 32
    rows_w = B * S // NW
    x3 = x.astype(jnp.int32).reshape(NW, rows_w // _C, _C)
    enc = encoding[:S, :] * (1.0 / math.sqrt(float(D)))
    enc2 = jnp.concatenate([enc, enc], axis=0)
    out = _build(B, S, D, V)(table, x3, enc2)
    return out.reshape(B, S, D)
